# Initial kernel scaffold; baseline (speedup 1.0000x reference)
import jax
import jax.numpy as jnp
from jax.experimental import pallas as pl


def kernel(x, edge_index, W1, b1, W2, b2, Wl, bl):
    raise NotImplementedError("write your pallas kernel here")


# speedup vs baseline: 20.4761x; 20.4761x over previous
import functools

import jax
import jax.numpy as jnp
from jax import lax
from jax.experimental import pallas as pl
from jax.experimental.pallas import tpu as pltpu
from jax.experimental.pallas import tpu_sc as plsc

N = 10000
D = 128
H = 128
C = 2
E = 320000

NC = 2
NS = 16
NW = NC * NS

NPAD = 10240
ROWS_PER_TILE = NPAD // NS
CHUNK = 128
CHUNKS = (E + NW * CHUNK - 1) // (NW * CHUNK)
EPT = CHUNKS * CHUNK
EPAD = EPT * NW


def _sc_mesh():
    return plsc.VectorSubcoreMesh(core_axis_name="c", subcore_axis_name="s",
                                  num_cores=NC, num_subcores=NS)


def _deg_body(dst_hbm, ones_hbm, z640_hbm, degp_hbm,
              deg_sh, idx_v, ones_v, z_v, sem):
    c = lax.axis_index("c")
    s = lax.axis_index("s")
    w = c * NS + s
    base = s * ROWS_PER_TILE
    pltpu.sync_copy(z640_hbm, z_v)
    pltpu.sync_copy(z_v, deg_sh.at[pl.ds(base, ROWS_PER_TILE)])
    pltpu.sync_copy(ones_hbm, ones_v)
    plsc.subcore_barrier()

    def step(j, _):
        pltpu.sync_copy(dst_hbm.at[w, j], idx_v)
        pltpu.sync_copy(ones_v, deg_sh.at[idx_v], add=True)
        return _

    lax.fori_loop(0, CHUNKS, step, 0)
    plsc.subcore_barrier()
    pltpu.sync_copy(deg_sh.at[pl.ds(base, ROWS_PER_TILE)], z_v)
    pltpu.sync_copy(z_v, degp_hbm.at[c, pl.ds(base, ROWS_PER_TILE)])


def _run_deg(dst3, ones128, z640):
    k = pl.kernel(
        _deg_body,
        out_type=jax.ShapeDtypeStruct((NC, NPAD), jnp.float32),
        mesh=_sc_mesh(),
        scratch_types=[
            pltpu.VMEM_SHARED((NPAD,), jnp.float32),
            pltpu.VMEM((CHUNK,), jnp.int32),
            pltpu.VMEM((CHUNK,), jnp.float32),
            pltpu.VMEM((ROWS_PER_TILE,), jnp.float32),
            pltpu.SemaphoreType.DMA,
        ],
    )
    return k(dst3, ones128, z640)


def _msg_body(hs_hbm, dinv_hbm, src3_hbm, dst3_hbm, z2d_hbm,
              accp_hbm, cp_hbm,
              acc_sh, cacc_sh,
              src_idx, dst_idx, rows_v, dvals, sem, sem2):
    c = lax.axis_index("c")
    s = lax.axis_index("s")
    w = c * NS + s
    base = s * ROWS_PER_TILE

    pltpu.sync_copy(src3_hbm.at[w], src_idx)
    pltpu.sync_copy(dst3_hbm.at[w], dst_idx)
    pltpu.sync_copy(z2d_hbm, rows_v)
    for k in range(8):
        dvals[pl.ds(k * 16, 16)] = jnp.zeros((16,), jnp.float32)
    for k in range(ROWS_PER_TILE // CHUNK):
        pltpu.sync_copy(rows_v, acc_sh.at[pl.ds(base + k * CHUNK, CHUNK)])
        pltpu.sync_copy(dvals, cacc_sh.at[pl.ds(base + k * CHUNK, CHUNK)])
    plsc.subcore_barrier()

    def step(j, _):
        gat = pltpu.async_copy(hs_hbm.at[src_idx.at[j]], rows_v, sem)
        pltpu.async_copy(dinv_hbm.at[dst_idx.at[j]], dvals, sem2).wait()
        pltpu.sync_copy(dvals, cacc_sh.at[src_idx.at[j]], add=True)
        gat.wait()
        pltpu.sync_copy(rows_v, acc_sh.at[dst_idx.at[j]], add=True)
        return _

    lax.fori_loop(0, CHUNKS, step, 0)
    plsc.subcore_barrier()

    for k in range(ROWS_PER_TILE // CHUNK):
        r = base + k * CHUNK
        pltpu.sync_copy(acc_sh.at[pl.ds(r, CHUNK)], rows_v)
        pltpu.sync_copy(rows_v, accp_hbm.at[c, pl.ds(r, CHUNK)])
        pltpu.sync_copy(cacc_sh.at[pl.ds(r, CHUNK)], dvals)
        pltpu.sync_copy(dvals, cp_hbm.at[c, pl.ds(r, CHUNK)])


def _run_msg(hs, dinv, src3, dst3, z2d):
    k = pl.kernel(
        _msg_body,
        out_type=(
            jax.ShapeDtypeStruct((NC, NPAD, H), jnp.float32),
            jax.ShapeDtypeStruct((NC, NPAD), jnp.float32),
        ),
        mesh=_sc_mesh(),
        scratch_types=[
            pltpu.VMEM_SHARED((NPAD, H), jnp.float32),
            pltpu.VMEM_SHARED((NPAD,), jnp.float32),
            pltpu.VMEM((CHUNKS, CHUNK), jnp.int32),
            pltpu.VMEM((CHUNKS, CHUNK), jnp.int32),
            pltpu.VMEM((CHUNK, H), jnp.float32),
            pltpu.VMEM((CHUNK,), jnp.float32),
            pltpu.SemaphoreType.DMA,
            pltpu.SemaphoreType.DMA,
        ],
    )
    return k(hs, dinv, src3, dst3, z2d)


BLK1 = 512


def _dense1_body(x_ref, w1_ref, degt_ref, hs_ref, dinv_ref):
    pid = pl.program_id(0)
    deg = degt_ref[:, 0:1] + degt_ref[:, 1:2] + 1.0
    row = pid * BLK1 + lax.broadcasted_iota(jnp.int32, (BLK1, 1), 0)
    dinv = jnp.where(row < N, lax.rsqrt(jnp.maximum(deg, 1.0)), 0.0)
    dinv_ref[...] = dinv
    h = jnp.dot(x_ref[...], w1_ref[...], preferred_element_type=jnp.float32)
    hs_ref[...] = h * dinv


def _run_dense1(xp, W1, degT):
    grid = (NPAD // BLK1,)
    return pl.pallas_call(
        _dense1_body,
        grid=grid,
        in_specs=[
            pl.BlockSpec((BLK1, D), lambda i: (i, 0)),
            pl.BlockSpec((D, H), lambda i: (0, 0)),
            pl.BlockSpec((BLK1, NC), lambda i: (i, 0)),
        ],
        out_specs=[
            pl.BlockSpec((BLK1, H), lambda i: (i, 0)),
            pl.BlockSpec((BLK1, 1), lambda i: (i, 0)),
        ],
        out_shape=[
            jax.ShapeDtypeStruct((NPAD, H), jnp.float32),
            jax.ShapeDtypeStruct((NPAD, 1), jnp.float32),
        ],
    )(xp, W1, degT)


BLK2 = 512


def _final_body(accp_ref, hs_ref, dinv_ref, cpt_ref, b1_ref, w2_ref, b2_ref,
                wl_ref, bl_ref, out_ref, g_ref):
    i = pl.program_id(0)

    @pl.when(i == 0)
    def _init():
        g_ref[...] = jnp.zeros_like(g_ref)

    acc = accp_ref[0] + accp_ref[1] + hs_ref[...]
    dinv = dinv_ref[...]
    h1 = jnp.maximum(dinv * acc + b1_ref[...], 0.0)
    cvec = dinv * (cpt_ref[:, 0:1] + cpt_ref[:, 1:2] + dinv)
    g_ref[...] += jnp.sum(h1 * cvec, axis=0, keepdims=True)

    @pl.when(i == pl.num_programs(0) - 1)
    def _fin():
        g = g_ref[...] * (1.0 / N)
        t = jnp.dot(g, w2_ref[...], preferred_element_type=jnp.float32)
        t = t + b2_ref[...]
        out_ref[...] = (
            jnp.dot(t, wl_ref[...], preferred_element_type=jnp.float32)
            + bl_ref[...])


def _run_final(accp, hs, dinv, cpT, b1, W2, b2, Wl, bl):
    grid = (NPAD // BLK2,)
    return pl.pallas_call(
        _final_body,
        grid=grid,
        in_specs=[
            pl.BlockSpec((NC, BLK2, H), lambda i: (0, i, 0)),
            pl.BlockSpec((BLK2, H), lambda i: (i, 0)),
            pl.BlockSpec((BLK2, 1), lambda i: (i, 0)),
            pl.BlockSpec((BLK2, NC), lambda i: (i, 0)),
            pl.BlockSpec((1, H), lambda i: (0, 0)),
            pl.BlockSpec((H, H), lambda i: (0, 0)),
            pl.BlockSpec((1, H), lambda i: (0, 0)),
            pl.BlockSpec((H, C), lambda i: (0, 0)),
            pl.BlockSpec((1, C), lambda i: (0, 0)),
        ],
        out_specs=pl.BlockSpec((1, C), lambda i: (0, 0)),
        out_shape=jax.ShapeDtypeStruct((1, C), jnp.float32),
        scratch_shapes=[pltpu.VMEM((1, H), jnp.float32)],
    )(accp, hs, dinv, cpT, b1, W2, b2, Wl, bl)


@functools.partial(jax.jit)
def kernel(x, edge_index, W1, b1, W2, b2, Wl, bl):
    ei = edge_index.astype(jnp.int32)
    pad = jnp.full((EPAD - E,), NPAD - 1, jnp.int32)
    src3 = jnp.concatenate([ei[0], pad]).reshape(NW, CHUNKS, CHUNK)
    dst3 = jnp.concatenate([ei[1], pad]).reshape(NW, CHUNKS, CHUNK)
    xp = jnp.pad(x.astype(jnp.float32), ((0, NPAD - N), (0, 0)))

    ones128 = jnp.ones((CHUNK,), jnp.float32)
    z640 = jnp.zeros((ROWS_PER_TILE,), jnp.float32)
    z2d = jnp.zeros((CHUNK, H), jnp.float32)

    degp = _run_deg(dst3, ones128, z640)
    hs, dinv2 = _run_dense1(xp, W1.astype(jnp.float32), degp.T)
    dinv1 = dinv2.reshape(NPAD)
    accp, cp = _run_msg(hs, dinv1, src3, dst3, z2d)
    out = _run_final(accp, hs, dinv2, cp.T,
                     b1.reshape(1, H).astype(jnp.float32),
                     W2.astype(jnp.float32),
                     b2.reshape(1, H).astype(jnp.float32),
                     Wl.astype(jnp.float32),
                     bl.reshape(1, C).astype(jnp.float32))
    return out
